# baseline (speedup 1.0000x reference)
import functools

import jax
import jax.numpy as jnp
from jax import lax
from jax.experimental import pallas as pl
from jax.experimental.pallas import tpu as pltpu
from jax.experimental.pallas import tpu_sc as plsc

V = 100000
H = 768
H2 = H // 2
P = 512
D_IMG = 2048
B = 128
LT = 512
LI = 196
EPS = 1e-12

NTOK_TOTAL = B * LT
NC, NS = 2, 16
NW = NC * NS
NCHUNK = 4
BC = B // NCHUNK
CTOK = BC * LT
TPW = CTOK // NW
CH = 32
NBUF = 2


def _sc_gather_pack(table, idx_flat):
    mesh = plsc.VectorSubcoreMesh(core_axis_name="c", subcore_axis_name="s")

    @functools.partial(
        pl.kernel,
        mesh=mesh,
        out_type=jax.ShapeDtypeStruct((CTOK, H2), jnp.uint32),
        scratch_types=[
            pltpu.VMEM((TPW,), jnp.int32),
            pltpu.VMEM((NBUF, CH, H), jnp.float32),
            pltpu.VMEM((NBUF, CH, H2), jnp.uint32),
            pltpu.SemaphoreType.DMA,
        ],
        compiler_params=pltpu.CompilerParams(needs_layout_passes=False),
    )
    def gather_k(idx_hbm, table_hbm, out_hbm, idx_v, rows_v, pack_v, gsem):
        wid = lax.axis_index("s") * NC + lax.axis_index("c")
        base = wid * TPW
        pltpu.sync_copy(idx_hbm.at[pl.ds(base, TPW)], idx_v)
        niter = TPW // CH

        for b in range(NBUF):
            pltpu.async_copy(
                table_hbm.at[idx_v.at[pl.ds(b * CH, CH)]], rows_v.at[b], gsem
            )

        def group(g, carry):
            for b in range(NBUF):
                i = g * NBUF + b
                off = i * CH
                pltpu.make_async_copy(
                    table_hbm.at[idx_v.at[pl.ds(off, CH)]], rows_v.at[b], gsem
                ).wait()

                @plsc.parallel_loop(0, CH, 1, unroll=1)
                def row(r):
                    for gr in range(H2 // 16):
                        sl = pl.ds(gr * 16, 16)
                        lo = rows_v[b, r, sl]
                        hi = rows_v[b, r, pl.ds(H2 + gr * 16, 16)]
                        pk = plsc.pack(lo, hi, format=plsc.PackFormat.INTERLEAVED)
                        pack_v[b, r, sl] = plsc.bitcast(pk, jnp.uint32)

                pltpu.sync_copy(pack_v.at[b], out_hbm.at[pl.ds(base + off, CH)])

                @pl.when(i + NBUF < niter)
                def _():
                    pltpu.async_copy(
                        table_hbm.at[idx_v.at[pl.ds((i + NBUF) * CH, CH)]],
                        rows_v.at[b], gsem,
                    )
            return carry

        lax.fori_loop(0, niter // NBUF, group, 0)

    return gather_k(idx_flat, table)


def _ln(x, g, b):
    m = jnp.mean(x, axis=-1, keepdims=True)
    v = jnp.mean(x * x, axis=-1, keepdims=True) - m * m
    return (x - m) * lax.rsqrt(v + EPS) * g + b


def _tc_chunk(prev_out, chunk, wrows, pos_t, seg_t, feat, pos_i, seg_i,
              pos_tab_t, pos_tab_i, tt_tab, img_W, img_b, img_ln_g, img_ln_b,
              ln_t_g, ln_t_b, ln_i_g, ln_i_b):
    b0 = chunk * BC

    def body(*refs):
        if prev_out is None:
            (wrows_ref, pos_t_ref, seg_t_ref, feat_ref, pos_i_ref, seg_i_ref,
             pos_tab_t_ref, pos_tab_i_ref, tt_ref, img_W_ref, img_b_ref,
             img_ln_g_ref, img_ln_b_ref, ln_t_g_ref, ln_t_b_ref, ln_i_g_ref,
             ln_i_b_ref, out_ref) = refs
        else:
            (_prev_ref, wrows_ref, pos_t_ref, seg_t_ref, feat_ref, pos_i_ref,
             seg_i_ref, pos_tab_t_ref, pos_tab_i_ref, tt_ref, img_W_ref,
             img_b_ref, img_ln_g_ref, img_ln_b_ref, ln_t_g_ref, ln_t_b_ref,
             ln_i_g_ref, ln_i_b_ref, out_ref) = refs
        tt0 = tt_ref[0]
        tt1 = tt_ref[1]

        w = wrows_ref[0]
        wlo = lax.bitcast_convert_type(w << 16, jnp.float32)
        whi = lax.bitcast_convert_type(w & jnp.uint32(0xFFFF0000), jnp.float32)
        wr = jnp.concatenate([wlo, whi], axis=1)
        pos = pos_t_ref[0, 0]
        oh = (pos[:, None]
              == lax.broadcasted_iota(jnp.int32, (LT, P), 1)).astype(jnp.bfloat16)
        pe = jnp.dot(oh, pos_tab_t_ref[...], preferred_element_type=jnp.float32)
        seg = seg_t_ref[0, 0][:, None] > 0
        t = wr + pe + jnp.where(seg, tt1, tt0)
        out_ref[0, :LT] = _ln(t, ln_t_g_ref[0], ln_t_b_ref[0])

        im = jnp.dot(feat_ref[0].astype(jnp.bfloat16), img_W_ref[...],
                     preferred_element_type=jnp.float32) + img_b_ref[0]
        im = _ln(im, img_ln_g_ref[0], img_ln_b_ref[0])
        posi = pos_i_ref[0, 0]
        ohi = (posi[:, None]
               == lax.broadcasted_iota(jnp.int32, (LI, P), 1)).astype(jnp.bfloat16)
        pei = jnp.dot(ohi, pos_tab_i_ref[...], preferred_element_type=jnp.float32)
        segi = seg_i_ref[0, 0][:, None] > 0
        im = im + pei + jnp.where(segi, tt1, tt0)
        out_ref[0, LT:] = _ln(im, ln_i_g_ref[0], ln_i_b_ref[0])

    row = lambda shape: pl.BlockSpec(shape, lambda b: (0,) * len(shape))
    off = lambda shape: pl.BlockSpec(shape, lambda b: (b0 + b, 0, 0))
    in_specs = [
        pl.BlockSpec((1, LT, H2), lambda b: (b, 0, 0)),
        off((1, 1, LT)),
        off((1, 1, LT)),
        off((1, LI, D_IMG)),
        off((1, 1, LI)),
        off((1, 1, LI)),
        row((P, H)),
        row((P, H)),
        row((2, H)),
        row((D_IMG, H)),
        row((1, H)),
        row((1, H)),
        row((1, H)),
        row((1, H)),
        row((1, H)),
        row((1, H)),
        row((1, H)),
    ]
    args = [wrows, pos_t, seg_t, feat, pos_i, seg_i, pos_tab_t, pos_tab_i,
            tt_tab, img_W, img_b, img_ln_g, img_ln_b, ln_t_g, ln_t_b,
            ln_i_g, ln_i_b]
    io_aliases = {}
    if prev_out is not None:
        in_specs = [pl.BlockSpec(memory_space=pl.ANY)] + in_specs
        args = [prev_out] + args
        io_aliases = {0: 0}
    return pl.pallas_call(
        body,
        grid=(BC,),
        in_specs=in_specs,
        out_specs=pl.BlockSpec((1, LT + LI, H), lambda b: (b0 + b, 0, 0)),
        out_shape=jax.ShapeDtypeStruct((B, LT + LI, H), jnp.float32),
        input_output_aliases=io_aliases,
    )(*args)


def kernel(input_ids_text, position_ids_text, segment_ids_text, image_feat,
           position_ids_image, segment_ids_image, word_emb, pos_emb_text,
           pos_emb_image, token_type_emb, img_W, img_b, img_ln_g, img_ln_b,
           ln_text_g, ln_text_b, ln_img_g, ln_img_b):
    ids = input_ids_text.reshape(NTOK_TOTAL)
    wrows = [_sc_gather_pack(word_emb, ids[c * CTOK:(c + 1) * CTOK])
             for c in range(NCHUNK)]
    r1 = lambda v: v.reshape(1, H)
    common = dict(
        pos_t=position_ids_text.reshape(B, 1, LT),
        seg_t=segment_ids_text.reshape(B, 1, LT),
        feat=image_feat,
        pos_i=position_ids_image.reshape(B, 1, LI),
        seg_i=segment_ids_image.reshape(B, 1, LI),
        pos_tab_t=pos_emb_text.astype(jnp.bfloat16),
        pos_tab_i=pos_emb_image.astype(jnp.bfloat16),
        tt_tab=token_type_emb,
        img_W=img_W.astype(jnp.bfloat16),
        img_b=r1(img_b), img_ln_g=r1(img_ln_g), img_ln_b=r1(img_ln_b),
        ln_t_g=r1(ln_text_g), ln_t_b=r1(ln_text_b),
        ln_i_g=r1(ln_img_g), ln_i_b=r1(ln_img_b),
    )
    out = None
    for c in range(NCHUNK):
        out = _tc_chunk(out, c, wrows[c].reshape(BC, LT, H2), **common)
    return out


# speedup vs baseline: 2.0139x; 2.0139x over previous
import functools

import jax
import jax.numpy as jnp
from jax import lax
from jax.experimental import pallas as pl
from jax.experimental.pallas import tpu as pltpu
from jax.experimental.pallas import tpu_sc as plsc

V = 100000
H = 768
H2 = H // 2
P = 512
D_IMG = 2048
B = 128
LT = 512
LI = 196
LO = LT + LI
EPS = 1e-12

NC, NS = 2, 16
NW = NC * NS
NCHUNK = 4
CLT = LT // NCHUNK
CTOK = CLT * B
TPW = CTOK // NW
CH = 32
NBUF = 2
TB = 16
IB = 8


def _sc_gather_pack(table, idx_flat):
    mesh = plsc.VectorSubcoreMesh(core_axis_name="c", subcore_axis_name="s")

    @functools.partial(
        pl.kernel,
        mesh=mesh,
        out_type=jax.ShapeDtypeStruct((CTOK, H2), jnp.uint32),
        scratch_types=[
            pltpu.VMEM((TPW,), jnp.int32),
            pltpu.VMEM((NBUF, CH, H), jnp.float32),
            pltpu.VMEM((NBUF, CH, H2), jnp.uint32),
            pltpu.SemaphoreType.DMA,
        ],
        compiler_params=pltpu.CompilerParams(needs_layout_passes=False),
    )
    def gather_k(idx_hbm, table_hbm, out_hbm, idx_v, rows_v, pack_v, gsem):
        wid = lax.axis_index("s") * NC + lax.axis_index("c")
        base = wid * TPW
        pltpu.sync_copy(idx_hbm.at[pl.ds(base, TPW)], idx_v)
        niter = TPW // CH

        for b in range(NBUF):
            pltpu.async_copy(
                table_hbm.at[idx_v.at[pl.ds(b * CH, CH)]], rows_v.at[b], gsem
            )

        def group(g, carry):
            for b in range(NBUF):
                i = g * NBUF + b
                off = i * CH
                pltpu.make_async_copy(
                    table_hbm.at[idx_v.at[pl.ds(off, CH)]], rows_v.at[b], gsem
                ).wait()

                @plsc.parallel_loop(0, CH, 1, unroll=1)
                def row(r):
                    for gr in range(H2 // 16):
                        sl = pl.ds(gr * 16, 16)
                        lo = rows_v[b, r, sl]
                        hi = rows_v[b, r, pl.ds(H2 + gr * 16, 16)]
                        pk = plsc.pack(lo, hi, format=plsc.PackFormat.INTERLEAVED)
                        pack_v[b, r, sl] = plsc.bitcast(pk, jnp.uint32)

                pltpu.sync_copy(pack_v.at[b], out_hbm.at[pl.ds(base + off, CH)])

                @pl.when(i + NBUF < niter)
                def _():
                    pltpu.async_copy(
                        table_hbm.at[idx_v.at[pl.ds((i + NBUF) * CH, CH)]],
                        rows_v.at[b], gsem,
                    )
            return carry

        lax.fori_loop(0, niter // NBUF, group, 0)

    return gather_k(idx_flat, table)


def _ln(x, g, b):
    m = jnp.mean(x, axis=-1, keepdims=True)
    v = jnp.mean(x * x, axis=-1, keepdims=True) - m * m
    return (x - m) * lax.rsqrt(v + EPS) * g + b


def _pos_tt(pos, seg, n, pos_tab_ref, tt_ref):
    oh = (pos[:, :, None]
          == lax.broadcasted_iota(jnp.int32, (n, B, P), 2)).astype(jnp.bfloat16)
    pe = lax.dot_general(oh, pos_tab_ref[...], (((2,), (0,)), ((), ())),
                         preferred_element_type=jnp.float32)
    return pe + jnp.where(seg[:, :, None] > 0, tt_ref[1], tt_ref[0])


def _tc_image(feat_t, pos_i, seg_i, pos_tab_i, tt_tab, img_W, img_b,
              img_ln_g, img_ln_b, ln_i_g, ln_i_b):
    def body(feat_ref, pos_i_ref, seg_i_ref, pos_tab_i_ref, tt_ref, img_W_ref,
             img_b_ref, img_ln_g_ref, img_ln_b_ref, ln_i_g_ref, ln_i_b_ref,
             out_ref):
        im = lax.dot_general(
            feat_ref[...].astype(jnp.bfloat16), img_W_ref[...],
            (((2,), (0,)), ((), ())),
            preferred_element_type=jnp.float32) + img_b_ref[0]
        im = _ln(im, img_ln_g_ref[0], img_ln_b_ref[0])
        im = im + _pos_tt(pos_i_ref[0], seg_i_ref[0], IB, pos_tab_i_ref, tt_ref)
        out_ref[...] = _ln(im, ln_i_g_ref[0], ln_i_b_ref[0])

    row = lambda shape: pl.BlockSpec(shape, lambda s: (0,) * len(shape))
    return pl.pallas_call(
        body,
        grid=(pl.cdiv(LI, IB),),
        in_specs=[
            pl.BlockSpec((IB, B, D_IMG), lambda s: (s, 0, 0)),
            pl.BlockSpec((1, IB, B), lambda s: (0, s, 0)),
            pl.BlockSpec((1, IB, B), lambda s: (0, s, 0)),
            row((P, H)),
            row((2, H)),
            row((D_IMG, H)),
            row((1, H)),
            row((1, H)),
            row((1, H)),
            row((1, H)),
            row((1, H)),
        ],
        out_specs=pl.BlockSpec((IB, B, H), lambda s: (LT // IB + s, 0, 0)),
        out_shape=jax.ShapeDtypeStruct((LO, B, H), jnp.float32),
    )(feat_t, pos_i, seg_i, pos_tab_i, tt_tab, img_W, img_b, img_ln_g,
      img_ln_b, ln_i_g, ln_i_b)


def _tc_text_chunk(prev_out, chunk, wrows, pos_t, seg_t, pos_tab_t, tt_tab,
                   ln_t_g, ln_t_b):
    c0 = chunk * CLT

    def body(_prev_ref, wrows_ref, pos_t_ref, seg_t_ref, pos_tab_t_ref,
             tt_ref, ln_t_g_ref, ln_t_b_ref, out_ref):
        w = wrows_ref[...]
        wlo = lax.bitcast_convert_type(w << 16, jnp.float32)
        whi = lax.bitcast_convert_type(w & jnp.uint32(0xFFFF0000), jnp.float32)
        t = jnp.concatenate([wlo, whi], axis=2)
        t = t + _pos_tt(pos_t_ref[0], seg_t_ref[0], TB, pos_tab_t_ref, tt_ref)
        out_ref[...] = _ln(t, ln_t_g_ref[0], ln_t_b_ref[0])

    row = lambda shape: pl.BlockSpec(shape, lambda s: (0,) * len(shape))
    return pl.pallas_call(
        body,
        grid=(CLT // TB,),
        in_specs=[
            pl.BlockSpec(memory_space=pl.ANY),
            pl.BlockSpec((TB, B, H2), lambda s: (s, 0, 0)),
            pl.BlockSpec((1, TB, B), lambda s: (0, c0 // TB + s, 0)),
            pl.BlockSpec((1, TB, B), lambda s: (0, c0 // TB + s, 0)),
            row((P, H)),
            row((2, H)),
            row((1, H)),
            row((1, H)),
        ],
        out_specs=pl.BlockSpec((TB, B, H), lambda s: (c0 // TB + s, 0, 0)),
        out_shape=jax.ShapeDtypeStruct((LO, B, H), jnp.float32),
        input_output_aliases={0: 0},
    )(prev_out, wrows, pos_t, seg_t, pos_tab_t, tt_tab, ln_t_g, ln_t_b)


def kernel(input_ids_text, position_ids_text, segment_ids_text, image_feat,
           position_ids_image, segment_ids_image, word_emb, pos_emb_text,
           pos_emb_image, token_type_emb, img_W, img_b, img_ln_g, img_ln_b,
           ln_text_g, ln_text_b, ln_img_g, ln_img_b):
    ids_t = input_ids_text.T.reshape(LT * B)
    wrows = [_sc_gather_pack(word_emb, ids_t[c * CTOK:(c + 1) * CTOK])
             for c in range(NCHUNK)]
    r1 = lambda v: v.reshape(1, H)
    out = _tc_image(
        image_feat.transpose(1, 0, 2),
        position_ids_image.T.reshape(1, LI, B),
        segment_ids_image.T.reshape(1, LI, B),
        pos_emb_image.astype(jnp.bfloat16), token_type_emb,
        img_W.astype(jnp.bfloat16), r1(img_b), r1(img_ln_g), r1(img_ln_b),
        r1(ln_img_g), r1(ln_img_b),
    )
    pos_t = position_ids_text.T.reshape(1, LT, B)
    seg_t = segment_ids_text.T.reshape(1, LT, B)
    pos_tab_t = pos_emb_text.astype(jnp.bfloat16)
    for c in range(NCHUNK):
        out = _tc_text_chunk(out, c, wrows[c].reshape(CLT, B, H2), pos_t,
                             seg_t, pos_tab_t, token_type_emb, r1(ln_text_g),
                             r1(ln_text_b))
    return out.transpose(1, 0, 2)


# speedup vs baseline: 2.0393x; 1.0127x over previous
import functools

import jax
import jax.numpy as jnp
from jax import lax
from jax.experimental import pallas as pl
from jax.experimental.pallas import tpu as pltpu
from jax.experimental.pallas import tpu_sc as plsc

V = 100000
H = 768
H2 = H // 2
P = 512
D_IMG = 2048
B = 128
LT = 512
LI = 196
LO = LT + LI
EPS = 1e-12

NC, NS = 2, 16
NW = NC * NS
NCHUNK = 4
CLT = LT // NCHUNK
CTOK = CLT * B
TPW = CTOK // NW
CH = 32
NBUF = 2
TB = 16
IB = 8


def _sc_gather_pack(table, idx_flat):
    mesh = plsc.VectorSubcoreMesh(core_axis_name="c", subcore_axis_name="s")

    @functools.partial(
        pl.kernel,
        mesh=mesh,
        out_type=jax.ShapeDtypeStruct((CTOK, H2), jnp.uint32),
        scratch_types=[
            pltpu.VMEM((TPW,), jnp.int32),
            pltpu.VMEM((NBUF, CH, H), jnp.float32),
            pltpu.VMEM((NBUF, CH, H2), jnp.uint32),
            pltpu.SemaphoreType.DMA,
        ],
        compiler_params=pltpu.CompilerParams(needs_layout_passes=False),
    )
    def gather_k(idx_hbm, table_hbm, out_hbm, idx_v, rows_v, pack_v, gsem):
        wid = lax.axis_index("s") * NC + lax.axis_index("c")
        base = wid * TPW
        pltpu.sync_copy(idx_hbm.at[pl.ds(base, TPW)], idx_v)
        niter = TPW // CH

        for b in range(NBUF):
            pltpu.async_copy(
                table_hbm.at[idx_v.at[pl.ds(b * CH, CH)]], rows_v.at[b], gsem
            )

        def group(g, carry):
            for b in range(NBUF):
                i = g * NBUF + b
                off = i * CH
                pltpu.make_async_copy(
                    table_hbm.at[idx_v.at[pl.ds(off, CH)]], rows_v.at[b], gsem
                ).wait()

                @plsc.parallel_loop(0, CH, 1, unroll=1)
                def row(r):
                    for gr in range(H2 // 16):
                        sl = pl.ds(gr * 16, 16)
                        lo = rows_v[b, r, sl]
                        hi = rows_v[b, r, pl.ds(H2 + gr * 16, 16)]
                        pk = plsc.pack(lo, hi, format=plsc.PackFormat.INTERLEAVED)
                        pack_v[b, r, sl] = plsc.bitcast(pk, jnp.uint32)

                pltpu.sync_copy(pack_v.at[b], out_hbm.at[pl.ds(base + off, CH)])

                @pl.when(i + NBUF < niter)
                def _():
                    pltpu.async_copy(
                        table_hbm.at[idx_v.at[pl.ds((i + NBUF) * CH, CH)]],
                        rows_v.at[b], gsem,
                    )
            return carry

        lax.fori_loop(0, niter // NBUF, group, 0)

    return gather_k(idx_flat, table)


def _ln(x, g, b):
    m = jnp.mean(x, axis=-1, keepdims=True)
    v = jnp.mean(x * x, axis=-1, keepdims=True) - m * m
    return (x - m) * lax.rsqrt(v + EPS) * g + b


def _pos_tt(pos, seg, n, pos_tab_ref, tt_ref):
    oh = (pos[:, :, None]
          == lax.broadcasted_iota(jnp.int32, (n, B, P), 2)).astype(jnp.bfloat16)
    pe = lax.dot_general(oh, pos_tab_ref[...], (((2,), (0,)), ((), ())),
                         preferred_element_type=jnp.float32)
    return pe + jnp.where(seg[:, :, None] > 0, tt_ref[1], tt_ref[0])


def _tc_image(feat_t, pos_i, seg_i, pos_tab_i, tt_tab, img_W, img_b,
              img_ln_g, img_ln_b, ln_i_g, ln_i_b):
    def body(feat_ref, pos_i_ref, seg_i_ref, pos_tab_i_ref, tt_ref, img_W_ref,
             img_b_ref, img_ln_g_ref, img_ln_b_ref, ln_i_g_ref, ln_i_b_ref,
             out_ref):
        im = lax.dot_general(
            feat_ref[...].astype(jnp.bfloat16),
            img_W_ref[...].astype(jnp.bfloat16),
            (((2,), (0,)), ((), ())),
            preferred_element_type=jnp.float32) + img_b_ref[0]
        im = _ln(im, img_ln_g_ref[0], img_ln_b_ref[0])
        im = im + _pos_tt(pos_i_ref[0], seg_i_ref[0], IB, pos_tab_i_ref, tt_ref)
        out_ref[...] = _ln(im, ln_i_g_ref[0], ln_i_b_ref[0])

    row = lambda shape: pl.BlockSpec(shape, lambda s: (0,) * len(shape))
    return pl.pallas_call(
        body,
        grid=(pl.cdiv(LI, IB),),
        in_specs=[
            pl.BlockSpec((IB, B, D_IMG), lambda s: (s, 0, 0)),
            pl.BlockSpec((1, IB, B), lambda s: (0, s, 0)),
            pl.BlockSpec((1, IB, B), lambda s: (0, s, 0)),
            row((P, H)),
            row((2, H)),
            row((D_IMG, H)),
            row((1, H)),
            row((1, H)),
            row((1, H)),
            row((1, H)),
            row((1, H)),
        ],
        out_specs=pl.BlockSpec((IB, B, H), lambda s: (LT // IB + s, 0, 0)),
        out_shape=jax.ShapeDtypeStruct((LO, B, H), jnp.float32),
    )(feat_t, pos_i, seg_i, pos_tab_i, tt_tab, img_W, img_b, img_ln_g,
      img_ln_b, ln_i_g, ln_i_b)


def _tc_text_chunk(prev_out, chunk, wrows, pos_t, seg_t, pos_tab_t, tt_tab,
                   ln_t_g, ln_t_b):
    c0 = chunk * CLT

    def body(_prev_ref, wrows_ref, pos_t_ref, seg_t_ref, pos_tab_t_ref,
             tt_ref, ln_t_g_ref, ln_t_b_ref, out_ref):
        w = wrows_ref[...]
        wlo = lax.bitcast_convert_type(w << 16, jnp.float32)
        whi = lax.bitcast_convert_type(w & jnp.uint32(0xFFFF0000), jnp.float32)
        t = jnp.concatenate([wlo, whi], axis=2)
        t = t + _pos_tt(pos_t_ref[0], seg_t_ref[0], TB, pos_tab_t_ref, tt_ref)
        out_ref[...] = _ln(t, ln_t_g_ref[0], ln_t_b_ref[0])

    row = lambda shape: pl.BlockSpec(shape, lambda s: (0,) * len(shape))
    return pl.pallas_call(
        body,
        grid=(CLT // TB,),
        in_specs=[
            pl.BlockSpec(memory_space=pl.ANY),
            pl.BlockSpec((TB, B, H2), lambda s: (s, 0, 0)),
            pl.BlockSpec((1, TB, B), lambda s: (0, c0 // TB + s, 0)),
            pl.BlockSpec((1, TB, B), lambda s: (0, c0 // TB + s, 0)),
            row((P, H)),
            row((2, H)),
            row((1, H)),
            row((1, H)),
        ],
        out_specs=pl.BlockSpec((TB, B, H), lambda s: (c0 // TB + s, 0, 0)),
        out_shape=jax.ShapeDtypeStruct((LO, B, H), jnp.float32),
        input_output_aliases={0: 0},
    )(prev_out, wrows, pos_t, seg_t, pos_tab_t, tt_tab, ln_t_g, ln_t_b)


def kernel(input_ids_text, position_ids_text, segment_ids_text, image_feat,
           position_ids_image, segment_ids_image, word_emb, pos_emb_text,
           pos_emb_image, token_type_emb, img_W, img_b, img_ln_g, img_ln_b,
           ln_text_g, ln_text_b, ln_img_g, ln_img_b):
    ids_t = input_ids_text.T.reshape(LT * B)
    wrows = [_sc_gather_pack(word_emb, ids_t[c * CTOK:(c + 1) * CTOK])
             for c in range(NCHUNK)]
    r1 = lambda v: v.reshape(1, H)
    out = _tc_image(
        image_feat.transpose(1, 0, 2),
        position_ids_image.T.reshape(1, LI, B),
        segment_ids_image.T.reshape(1, LI, B),
        pos_emb_image.astype(jnp.bfloat16), token_type_emb,
        img_W, r1(img_b), r1(img_ln_g), r1(img_ln_b),
        r1(ln_img_g), r1(ln_img_b),
    )
    pos_t = position_ids_text.T.reshape(1, LT, B)
    seg_t = segment_ids_text.T.reshape(1, LT, B)
    pos_tab_t = pos_emb_text.astype(jnp.bfloat16)
    for c in range(NCHUNK):
        out = _tc_text_chunk(out, c, wrows[c].reshape(CLT, B, H2), pos_t,
                             seg_t, pos_tab_t, token_type_emb, r1(ln_text_g),
                             r1(ln_text_b))
    return out.transpose(1, 0, 2)
